# baseline (speedup 1.0000x reference)
import dataclasses

import jax
import jax.numpy as jnp
from jax import lax
from jax.experimental import pallas as pl
from jax.experimental.pallas import tpu as pltpu
from jax.experimental.pallas import tpu_sc as plsc

N = 10000
E = 160000
D = 256
H = 512
DE = 16

CHUNK = 128
NCHUNKS = E // CHUNK
NW = 32
NP = 2
NHALF = N // NP
ACC_ROWS = 5120
DUMMY = 5100
ZROWS = 320

_f32 = jnp.float32
_bf16 = jnp.bfloat16


def _sc_mesh():
    return plsc.VectorSubcoreMesh(core_axis_name="c", subcore_axis_name="s")


def _sc_params():
    cp = pltpu.CompilerParams()
    if "needs_layout_passes" in pltpu.CompilerParams.__dataclass_fields__:
        cp = dataclasses.replace(cp, needs_layout_passes=False)
    return cp


def _gather_body(hpk_hbm, cp4_hbm, row_hbm, col_hbm,
                 hr_out, hc_out, cdr_out,
                 idxr_v, idxc_v, hr_v, hc_v, cdr_v, coord_v,
                 s1, s2, s3):
    c = lax.axis_index("c")
    s = lax.axis_index("s")
    w = s * 2 + c

    pltpu.async_copy(cp4_hbm, coord_v, s3).wait()
    zero16 = jnp.zeros((16,), _f32)
    for t in (5, 6, 7):
        tcol = jnp.full((16,), t, jnp.int32)

        @pl.loop(0, CHUNK // 16)
        def _(j):
            ridx = j * 16 + lax.iota(jnp.int32, 16)
            plsc.store_scatter(cdr_v, [ridx, tcol], zero16)

    @pl.loop(0, 40)
    def _(k):
        chunk = w + NW * k

        @pl.when(chunk < NCHUNKS)
        def _():
            base = chunk * CHUNK
            pltpu.sync_copy(row_hbm.at[pl.ds(base, CHUNK)], idxr_v)
            pltpu.sync_copy(col_hbm.at[pl.ds(base, CHUNK)], idxc_v)
            d1 = pltpu.async_copy(hpk_hbm.at[idxr_v], hr_v, s1)
            d2 = pltpu.async_copy(hpk_hbm.at[idxc_v], hc_v, s2)

            @pl.loop(0, CHUNK // 16)
            def _(j):
                r16 = idxr_v[pl.ds(j * 16, 16)] * 3
                c16 = idxc_v[pl.ds(j * 16, 16)] * 3
                ridx = j * 16 + lax.iota(jnp.int32, 16)
                rad = jnp.zeros((16,), _f32)
                for t in range(3):
                    tcol = jnp.full((16,), t, jnp.int32)
                    a = plsc.load_gather(coord_v, [r16 + t])
                    b = plsc.load_gather(coord_v, [c16 + t])
                    dv = a - b
                    plsc.store_scatter(cdr_v, [ridx, tcol], dv)
                    rad = rad + dv * dv
                plsc.store_scatter(cdr_v, [ridx, jnp.full((16,), 4, jnp.int32)],
                                   rad)

            d1.wait()
            d2.wait()
            pltpu.sync_copy(hr_v, hr_out.at[pl.ds(base, CHUNK)])
            pltpu.sync_copy(hc_v, hc_out.at[pl.ds(base, CHUNK)])
            pltpu.sync_copy(cdr_v, cdr_out.at[pl.ds(base, CHUNK)])


def _sc_gather(hpk, cpad4, row, col):
    out_type = (
        jax.ShapeDtypeStruct((E, D // 2), jnp.int32),
        jax.ShapeDtypeStruct((E, D // 2), jnp.int32),
        jax.ShapeDtypeStruct((E, 8), _f32),
    )
    scratch = [
        pltpu.VMEM((CHUNK,), jnp.int32),
        pltpu.VMEM((CHUNK,), jnp.int32),
        pltpu.VMEM((CHUNK, D // 2), jnp.int32),
        pltpu.VMEM((CHUNK, D // 2), jnp.int32),
        pltpu.VMEM((CHUNK, 8), _f32),
        pltpu.VMEM((3 * N,), _f32),
        pltpu.SemaphoreType.DMA,
        pltpu.SemaphoreType.DMA,
        pltpu.SemaphoreType.DMA,
    ]
    fn = pl.kernel(_gather_body, out_type=out_type, mesh=_sc_mesh(),
                   scratch_types=scratch, compiler_params=_sc_params())
    return fn(hpk, cpad4, row, col)


NRANGE = 320
CAP = 5632
GROWS = NW * CAP
SCHUNK = 2000
LDUMMY = NRANGE


GCH = 64
NGCH = CAP // GCH


def _regroup_body(row_hbm, ef_hbm, t16_hbm,
                  gef_hbm, gt_hbm, lid_hbm,
                  rbuf_v, ids_v, lid_v, gefa_v, gefb_v, gta_v, gtb_v,
                  cnt_s, sga, sgb, swa, swb):
    c = lax.axis_index("c")
    s = lax.axis_index("s")
    w = s * 2 + c
    lo = w * NRANGE
    iota16 = lax.iota(jnp.int32, 16)

    zid = jnp.zeros((16,), jnp.int32)
    ldm = jnp.full((16,), LDUMMY, jnp.int32)

    @pl.loop(0, CAP // 16)
    def _(i):
        ids_v[pl.ds(i * 16, 16)] = zid
        lid_v[pl.ds(i * 16, 16)] = ldm

    cnt_s[0] = 0

    @pl.loop(0, E // SCHUNK)
    def _(k):
        pltpu.sync_copy(row_hbm.at[pl.ds(k * SCHUNK, SCHUNK)], rbuf_v)

        @pl.loop(0, SCHUNK // 16)
        def _(g):
            r16 = rbuf_v[pl.ds(g * 16, 16)]
            l16 = r16 - lo
            m = (l16 >= 0) & (l16 < NRANGE)
            eid = (k * SCHUNK + g * 16) + iota16
            cnt = cnt_s[0]
            plsc.store_compressed(ids_v.at[pl.ds(cnt, 16)], eid, mask=m)
            plsc.store_compressed(lid_v.at[pl.ds(cnt, 16)], l16, mask=m)
            cnt_s[0] = cnt + jnp.sum(m.astype(jnp.int32))

    pltpu.sync_copy(lid_v, lid_hbm.at[pl.ds(w * CAP, CAP)])

    def gath(k, buf_ef, buf_t, sem):
        iv = ids_v.at[pl.ds(k * GCH, GCH)]
        pltpu.async_copy(ef_hbm.at[iv], buf_ef, sem)
        pltpu.async_copy(t16_hbm.at[iv], buf_t, sem)

    def wr(k, buf_ef, buf_t, sem):
        base = w * CAP + k * GCH
        pltpu.async_copy(buf_ef, gef_hbm.at[pl.ds(base, GCH)], sem)
        pltpu.async_copy(buf_t, gt_hbm.at[pl.ds(base, GCH)], sem)

    def drain2(src, dst, sem):
        pltpu.make_async_copy(src, dst, sem).wait()

    gath(0, gefa_v, gta_v, sga)
    gath(1, gefb_v, gtb_v, sgb)

    @pl.loop(0, NGCH // 2)
    def _(i):
        k = 2 * i
        drain2(ef_hbm.at[pl.ds(0, GCH)], gefa_v, sga)
        drain2(t16_hbm.at[pl.ds(0, GCH)], gta_v, sga)
        wr(k, gefa_v, gta_v, swa)
        drain2(ef_hbm.at[pl.ds(0, GCH)], gefb_v, sgb)
        drain2(t16_hbm.at[pl.ds(0, GCH)], gtb_v, sgb)
        wr(k + 1, gefb_v, gtb_v, swb)

        @pl.when(k + 2 < NGCH)
        def _():
            drain2(gefa_v, gef_hbm.at[pl.ds(0, GCH)], swa)
            drain2(gta_v, gt_hbm.at[pl.ds(0, GCH)], swa)
            gath(k + 2, gefa_v, gta_v, sga)
            drain2(gefb_v, gef_hbm.at[pl.ds(0, GCH)], swb)
            drain2(gtb_v, gt_hbm.at[pl.ds(0, GCH)], swb)
            gath(k + 3, gefb_v, gtb_v, sgb)

    drain2(gefa_v, gef_hbm.at[pl.ds(0, GCH)], swa)
    drain2(gta_v, gt_hbm.at[pl.ds(0, GCH)], swa)
    drain2(gefb_v, gef_hbm.at[pl.ds(0, GCH)], swb)
    drain2(gtb_v, gt_hbm.at[pl.ds(0, GCH)], swb)


def _sc_regroup(row, ef, t16):
    out_type = (
        jax.ShapeDtypeStruct((GROWS, H), _f32),
        jax.ShapeDtypeStruct((GROWS, 128), _f32),
        jax.ShapeDtypeStruct((GROWS,), jnp.int32),
    )
    scratch = [
        pltpu.VMEM((SCHUNK,), jnp.int32),
        pltpu.VMEM((CAP,), jnp.int32),
        pltpu.VMEM((CAP,), jnp.int32),
        pltpu.VMEM((GCH, H), _f32),
        pltpu.VMEM((GCH, H), _f32),
        pltpu.VMEM((GCH, 128), _f32),
        pltpu.VMEM((GCH, 128), _f32),
        pltpu.SMEM((1,), jnp.int32),
        pltpu.SemaphoreType.DMA,
        pltpu.SemaphoreType.DMA,
        pltpu.SemaphoreType.DMA,
        pltpu.SemaphoreType.DMA,
    ]
    fn = pl.kernel(_regroup_body, out_type=out_type, mesh=_sc_mesh(),
                   scratch_types=scratch, compiler_params=_sc_params())
    return fn(row, ef, t16)


AGG_B = 512
AGG_K = CAP // AGG_B
AGG_R = 384


def _agg_kernel(lid_ref, gef_ref, gt_ref, nagg_ref, cagg_ref):
    k = pl.program_id(1)
    l2 = lid_ref[...].reshape(1, AGG_B)
    lb = jnp.broadcast_to(l2, (AGG_R, AGG_B))
    ohT = (lb == lax.broadcasted_iota(jnp.int32, (AGG_R, AGG_B), 0))
    ohT = ohT.astype(_bf16)
    c1 = jnp.dot(ohT, gef_ref[...].astype(_bf16),
                 preferred_element_type=_f32)
    c2 = jnp.dot(ohT, gt_ref[...].astype(_bf16),
                 preferred_element_type=_f32)

    @pl.when(k == 0)
    def _():
        nagg_ref[...] = jnp.zeros_like(nagg_ref)
        cagg_ref[...] = jnp.zeros_like(cagg_ref)

    nagg_ref[...] += c1
    cagg_ref[...] += c2


def _tc_agg(lid3, gef, gt):
    grid = (NW, AGG_K)
    return pl.pallas_call(
        _agg_kernel,
        grid=grid,
        in_specs=[
            pl.BlockSpec((1, 1, AGG_B), lambda w, k: (w * AGG_K + k, 0, 0)),
            pl.BlockSpec((AGG_B, H), lambda w, k: (w * AGG_K + k, 0)),
            pl.BlockSpec((AGG_B, 128), lambda w, k: (w * AGG_K + k, 0)),
        ],
        out_specs=[
            pl.BlockSpec((AGG_R, H), lambda w, k: (w, 0)),
            pl.BlockSpec((AGG_R, 128), lambda w, k: (w, 0)),
        ],
        out_shape=[
            jax.ShapeDtypeStruct((NW * AGG_R, H), _f32),
            jax.ShapeDtypeStruct((NW * AGG_R, 128), _f32),
        ],
    )(lid3, gef, gt)


def _silu(x):
    return x * jax.nn.sigmoid(x)


def _unpack_bf16(x):
    lo = lax.bitcast_convert_type(x << 16, _f32)
    hi = lax.bitcast_convert_type(x & jnp.int32(-65536), _f32)
    return lo, hi


def _edge_kernel(hr_ref, hc_ref, cdr_ref, ea_ref,
                 w1h_ref, w1er_ref, w2_ref, wc1_ref, wc2_ref, bias_ref,
                 ef_ref, t16_ref):
    lor, hir = _unpack_bf16(hr_ref[...])
    loc, hic = _unpack_bf16(hc_ref[...])
    hcat = jnp.concatenate([lor, hir, loc, hic], axis=1).astype(_bf16)
    cd = cdr_ref[...]
    radial = cd[:, 4:5]
    ea_ext = jnp.concatenate(
        [ea_ref[...], jnp.broadcast_to(radial, (radial.shape[0], 8))],
        axis=1).astype(_bf16)
    a = jnp.dot(hcat, w1h_ref[...], preferred_element_type=_f32)
    a += jnp.dot(ea_ext, w1er_ref[...], preferred_element_type=_f32)
    a += bias_ref[0:1, :]
    m = _silu(a).astype(_bf16)
    ef = jnp.dot(m, w2_ref[...], preferred_element_type=_f32)
    ef = _silu(ef + bias_ref[1:2, :])
    ef_ref[...] = ef
    cm = _silu(jnp.dot(ef.astype(_bf16), wc1_ref[...],
                       preferred_element_type=_f32) + bias_ref[2:3, :])
    scale = jnp.dot(cm.astype(_bf16), wc2_ref[...],
                    preferred_element_type=_f32)[:, 0:1]
    t8 = cd * scale
    t16_ref[...] = (jnp.concatenate(
        [t8, jnp.zeros((t8.shape[0], 120), _f32)], axis=1)
        + bias_ref[3:4, 0:128])


def _tc_edge(hr, hc, cdr, ea, w1h, w1er, w2, wc1, wc2p, bias):
    B = 1000
    grid = (E // B,)
    return pl.pallas_call(
        _edge_kernel,
        grid=grid,
        in_specs=[
            pl.BlockSpec((B, D // 2), lambda i: (i, 0)),
            pl.BlockSpec((B, D // 2), lambda i: (i, 0)),
            pl.BlockSpec((B, 8), lambda i: (i, 0)),
            pl.BlockSpec((B, DE), lambda i: (i, 0)),
            pl.BlockSpec((H, H), lambda i: (0, 0)),
            pl.BlockSpec((24, H), lambda i: (0, 0)),
            pl.BlockSpec((H, H), lambda i: (0, 0)),
            pl.BlockSpec((H, H), lambda i: (0, 0)),
            pl.BlockSpec((H, 128), lambda i: (0, 0)),
            pl.BlockSpec((8, H), lambda i: (0, 0)),
        ],
        out_specs=[
            pl.BlockSpec((B, H), lambda i: (i, 0)),
            pl.BlockSpec((B, 128), lambda i: (i, 0)),
        ],
        out_shape=[
            jax.ShapeDtypeStruct((E, H), _f32),
            jax.ShapeDtypeStruct((E, 128), _f32),
        ],
    )(hr, hc, cdr, ea, w1h, w1er, w2, wc1, wc2p, bias)


def _node_kernel(h_ref, nagg_ref, cagg_ref, cpad_ref,
                 w3a_ref, w3b_ref, w4_ref, bias_ref,
                 hout_ref, cout_ref):
    hb = h_ref[...].astype(_bf16)
    nb = nagg_ref[...].astype(_bf16)
    nhid = _silu(jnp.dot(hb, w3a_ref[...], preferred_element_type=_f32)
                 + jnp.dot(nb, w3b_ref[...], preferred_element_type=_f32)
                 + bias_ref[0:1, :])
    nout = jnp.dot(nhid.astype(_bf16), w4_ref[...],
                   preferred_element_type=_f32) + bias_ref[1:2, 0:D]
    hout_ref[...] = h_ref[...] + nout
    cagg = cagg_ref[...][:, 0:16]
    cnt = jnp.clip(cagg[:, 3:4], 1.0, None)
    cout_ref[...] = cpad_ref[...] + cagg / cnt


def _tc_node(h, nagg, cagg, cpad, w3a, w3b, w4, bias2):
    B = 1000
    grid = (N // B,)
    return pl.pallas_call(
        _node_kernel,
        grid=grid,
        in_specs=[
            pl.BlockSpec((B, D), lambda i: (i, 0)),
            pl.BlockSpec((B, H), lambda i: (i, 0)),
            pl.BlockSpec((B, 128), lambda i: (i, 0)),
            pl.BlockSpec((B, 16), lambda i: (i, 0)),
            pl.BlockSpec((D, H), lambda i: (0, 0)),
            pl.BlockSpec((H, H), lambda i: (0, 0)),
            pl.BlockSpec((H, D), lambda i: (0, 0)),
            pl.BlockSpec((8, H), lambda i: (0, 0)),
        ],
        out_specs=[
            pl.BlockSpec((B, D), lambda i: (i, 0)),
            pl.BlockSpec((B, 16), lambda i: (i, 0)),
        ],
        out_shape=[
            jax.ShapeDtypeStruct((N, D), _f32),
            jax.ShapeDtypeStruct((N, 16), _f32),
        ],
    )(h, nagg, cagg, cpad, w3a, w3b, w4, bias2)


def kernel(h, edge_index, coord, edge_attr,
           W1, b1, W2, b2, W3, b3, W4, b4, Wc1, bc1, Wc2):
    row = edge_index[0]
    col = edge_index[1]
    cpad = jnp.pad(coord, ((0, 0), (0, 13)))
    cflat = coord.reshape(3 * N)
    hpk = lax.bitcast_convert_type(
        h.astype(_bf16).reshape(N, D // 2, 2), jnp.int32)

    wr, wc = W1[:D], W1[D:2 * D]
    w1h = jnp.concatenate([wr[0::2], wr[1::2], wc[0::2], wc[1::2]],
                          axis=0).astype(_bf16)
    w1er = jnp.zeros((24, H), _f32).at[:DE].set(W1[2 * D + 1:]) \
        .at[DE].set(W1[2 * D]).astype(_bf16)
    bias = jnp.zeros((8, H), _f32).at[0].set(b1).at[1].set(b2) \
        .at[2].set(bc1).at[3, 3].set(1.0)
    wc2p = jnp.zeros((H, 128), _f32).at[:, 0].set(Wc2[:, 0]).astype(_bf16)
    w3a = W3[:D].astype(_bf16)
    w3b = W3[D:].astype(_bf16)
    w4 = W4.astype(_bf16)
    bias2 = jnp.zeros((8, H), _f32).at[0].set(b3).at[1, :D].set(b4)

    hr, hc, cdr = _sc_gather(hpk, cflat, row, col)
    ef, t16 = _tc_edge(hr, hc, cdr, edge_attr,
                       w1h, w1er, W2.astype(_bf16), Wc1.astype(_bf16),
                       wc2p, bias)
    gef, gt, lid = _sc_regroup(row, ef, t16)
    lid3 = lid.reshape(NW * AGG_K, 1, AGG_B)
    nagg_pad, cagg_pad = _tc_agg(lid3, gef, gt)
    nagg = nagg_pad.reshape(NW, AGG_R, H)[:, :NRANGE].reshape(NW * NRANGE, H)
    cagg = cagg_pad.reshape(NW, AGG_R, 128)[:, :NRANGE]
    cagg = cagg.reshape(NW * NRANGE, 128)
    h_out, cout16 = _tc_node(h, nagg[:N], cagg[:N], cpad,
                             w3a, w3b, w4, bias2)
    return h_out, cout16[:, :3], edge_attr


# speedup vs baseline: 1.0545x; 1.0545x over previous
import dataclasses

import jax
import jax.numpy as jnp
from jax import lax
from jax.experimental import pallas as pl
from jax.experimental.pallas import tpu as pltpu
from jax.experimental.pallas import tpu_sc as plsc

N = 10000
E = 160000
D = 256
H = 512
DE = 16

CHUNK = 128
NCHUNKS = E // CHUNK
NW = 32
NP = 2
NHALF = N // NP
ACC_ROWS = 5120
DUMMY = 5100
ZROWS = 320

_f32 = jnp.float32
_bf16 = jnp.bfloat16


def _sc_mesh():
    return plsc.VectorSubcoreMesh(core_axis_name="c", subcore_axis_name="s")


def _sc_params():
    cp = pltpu.CompilerParams()
    if "needs_layout_passes" in pltpu.CompilerParams.__dataclass_fields__:
        cp = dataclasses.replace(cp, needs_layout_passes=False)
    return cp


def _gather_body(hpk_hbm, cp4_hbm, row_hbm, col_hbm,
                 hr_out, hc_out, cdr_out,
                 idxr_v, idxc_v, hr_v, hc_v, cdr_v, coord_v,
                 s1, s2, s3):
    c = lax.axis_index("c")
    s = lax.axis_index("s")
    w = s * 2 + c

    pltpu.async_copy(cp4_hbm, coord_v, s3).wait()
    zero16 = jnp.zeros((16,), _f32)
    for t in (5, 6, 7):
        tcol = jnp.full((16,), t, jnp.int32)

        @pl.loop(0, CHUNK // 16)
        def _(j):
            ridx = j * 16 + lax.iota(jnp.int32, 16)
            plsc.store_scatter(cdr_v, [ridx, tcol], zero16)

    @pl.loop(0, 40)
    def _(k):
        chunk = w + NW * k

        @pl.when(chunk < NCHUNKS)
        def _():
            base = chunk * CHUNK
            pltpu.sync_copy(row_hbm.at[pl.ds(base, CHUNK)], idxr_v)
            pltpu.sync_copy(col_hbm.at[pl.ds(base, CHUNK)], idxc_v)
            d1 = pltpu.async_copy(hpk_hbm.at[idxr_v], hr_v, s1)
            d2 = pltpu.async_copy(hpk_hbm.at[idxc_v], hc_v, s2)

            @pl.loop(0, CHUNK // 16)
            def _(j):
                r16 = idxr_v[pl.ds(j * 16, 16)] * 3
                c16 = idxc_v[pl.ds(j * 16, 16)] * 3
                ridx = j * 16 + lax.iota(jnp.int32, 16)
                rad = jnp.zeros((16,), _f32)
                for t in range(3):
                    tcol = jnp.full((16,), t, jnp.int32)
                    a = plsc.load_gather(coord_v, [r16 + t])
                    b = plsc.load_gather(coord_v, [c16 + t])
                    dv = a - b
                    plsc.store_scatter(cdr_v, [ridx, tcol], dv)
                    rad = rad + dv * dv
                plsc.store_scatter(cdr_v, [ridx, jnp.full((16,), 4, jnp.int32)],
                                   rad)

            d1.wait()
            d2.wait()
            pltpu.sync_copy(hr_v, hr_out.at[pl.ds(base, CHUNK)])
            pltpu.sync_copy(hc_v, hc_out.at[pl.ds(base, CHUNK)])
            pltpu.sync_copy(cdr_v, cdr_out.at[pl.ds(base, CHUNK)])


def _sc_gather(hpk, cpad4, row, col):
    out_type = (
        jax.ShapeDtypeStruct((E, D // 2), jnp.int32),
        jax.ShapeDtypeStruct((E, D // 2), jnp.int32),
        jax.ShapeDtypeStruct((E, 8), _f32),
    )
    scratch = [
        pltpu.VMEM((CHUNK,), jnp.int32),
        pltpu.VMEM((CHUNK,), jnp.int32),
        pltpu.VMEM((CHUNK, D // 2), jnp.int32),
        pltpu.VMEM((CHUNK, D // 2), jnp.int32),
        pltpu.VMEM((CHUNK, 8), _f32),
        pltpu.VMEM((3 * N,), _f32),
        pltpu.SemaphoreType.DMA,
        pltpu.SemaphoreType.DMA,
        pltpu.SemaphoreType.DMA,
    ]
    fn = pl.kernel(_gather_body, out_type=out_type, mesh=_sc_mesh(),
                   scratch_types=scratch, compiler_params=_sc_params())
    return fn(hpk, cpad4, row, col)


NRANGE = 320
CAP = 5632
GROWS = NW * CAP
SCHUNK = 2000
LDUMMY = NRANGE


GCH = 32
NGCH = CAP // GCH
CAP2 = 832
SLICE = E // 16


def _regroup_body(row_hbm, ef_hbm, t16_hbm,
                  gef_hbm, gt_hbm, lid_hbm,
                  rbuf_v, ids_v, lid_v, binid_v, binlid_v, off_v,
                  mids_v, mlids_v, gefa_v, gefb_v, gta_v, gtb_v,
                  stag_ids, stag_lids, stag_cnt,
                  cnt_s, sd, sga, sgb, swa, swb):
    c = lax.axis_index("c")
    s = lax.axis_index("s")
    w = c * 16 + s
    iota16 = lax.iota(jnp.int32, 16)
    ones16 = jnp.ones((16,), jnp.int32)

    zid = jnp.zeros((16,), jnp.int32)
    ldm = jnp.full((16,), LDUMMY, jnp.int32)

    @pl.loop(0, CAP // 16)
    def _(i):
        ids_v[pl.ds(i * 16, 16)] = zid
        lid_v[pl.ds(i * 16, 16)] = ldm

    off_v[pl.ds(0, 16)] = jnp.zeros((16,), jnp.int32)

    @pl.loop(0, SLICE // SCHUNK)
    def _(k):
        pltpu.sync_copy(row_hbm.at[pl.ds(s * SLICE + k * SCHUNK, SCHUNK)],
                        rbuf_v)

        @pl.loop(0, SCHUNK // 16)
        def _(g):
            r16 = rbuf_v[pl.ds(g * 16, 16)]
            rel = r16 - c * (16 * NRANGE)
            m = (rel >= 0) & (rel < 16 * NRANGE)
            b16 = jnp.clip(rel // NRANGE, 0, 15)
            cnt1, lastm = plsc.scan_count(b16, mask=m)
            basev = plsc.load_gather(off_v, [b16])
            addr = b16 * CAP2 + basev + (cnt1 - 1)
            eid = (s * SLICE + k * SCHUNK + g * 16) + iota16
            plsc.store_scatter(binid_v, [addr], eid, mask=m)
            plsc.store_scatter(binlid_v, [addr], rel - b16 * NRANGE, mask=m)
            plsc.addupdate_scatter(off_v, [b16], cnt1, mask=m & lastm)

    @pl.loop(0, 16)
    def _(r):
        pltpu.sync_copy(binid_v.at[pl.ds(r * CAP2, CAP2)],
                        stag_ids.at[pl.ds((s * 16 + r) * CAP2, CAP2)])
        pltpu.sync_copy(binlid_v.at[pl.ds(r * CAP2, CAP2)],
                        stag_lids.at[pl.ds((s * 16 + r) * CAP2, CAP2)])

    pltpu.sync_copy(off_v, stag_cnt.at[pl.ds(s * 16, 16)])
    plsc.subcore_barrier()

    pltpu.async_copy(stag_cnt, cnt_s.at[pl.ds(0, 256)], sd).wait()
    cum_init = 0
    cnt_s[256] = cum_init
    for t in range(16):
        pltpu.sync_copy(stag_ids.at[pl.ds((t * 16 + s) * CAP2, CAP2)], mids_v)
        pltpu.sync_copy(stag_lids.at[pl.ds((t * 16 + s) * CAP2, CAP2)],
                        mlids_v)
        cnt_t = cnt_s[t * 16 + s]
        cum = cnt_s[256]

        @pl.loop(0, CAP2 // 16)
        def _(j):
            o = j * 16

            @pl.when(o < cnt_t)
            def _():
                mk = (o + iota16) < cnt_t
                plsc.store_compressed(ids_v.at[pl.ds(cum + o, 16)],
                                      mids_v[pl.ds(o, 16)], mask=mk)
                plsc.store_compressed(lid_v.at[pl.ds(cum + o, 16)],
                                      mlids_v[pl.ds(o, 16)], mask=mk)

        cnt_s[256] = cum + cnt_t

    pltpu.sync_copy(lid_v, lid_hbm.at[pl.ds(w * CAP, CAP)])

    def gath(k, buf_ef, buf_t, sem):
        iv = ids_v.at[pl.ds(k * GCH, GCH)]
        pltpu.async_copy(ef_hbm.at[iv], buf_ef, sem)
        pltpu.async_copy(t16_hbm.at[iv], buf_t, sem)

    def wr(k, buf_ef, buf_t, sem):
        base = w * CAP + k * GCH
        pltpu.async_copy(buf_ef, gef_hbm.at[pl.ds(base, GCH)], sem)
        pltpu.async_copy(buf_t, gt_hbm.at[pl.ds(base, GCH)], sem)

    def drain2(src, dst, sem):
        pltpu.make_async_copy(src, dst, sem).wait()

    gath(0, gefa_v, gta_v, sga)
    gath(1, gefb_v, gtb_v, sgb)

    @pl.loop(0, NGCH // 2)
    def _(i):
        k = 2 * i
        drain2(ef_hbm.at[pl.ds(0, GCH)], gefa_v, sga)
        drain2(t16_hbm.at[pl.ds(0, GCH)], gta_v, sga)
        wr(k, gefa_v, gta_v, swa)
        drain2(ef_hbm.at[pl.ds(0, GCH)], gefb_v, sgb)
        drain2(t16_hbm.at[pl.ds(0, GCH)], gtb_v, sgb)
        wr(k + 1, gefb_v, gtb_v, swb)

        @pl.when(k + 2 < NGCH)
        def _():
            drain2(gefa_v, gef_hbm.at[pl.ds(0, GCH)], swa)
            drain2(gta_v, gt_hbm.at[pl.ds(0, GCH)], swa)
            gath(k + 2, gefa_v, gta_v, sga)
            drain2(gefb_v, gef_hbm.at[pl.ds(0, GCH)], swb)
            drain2(gtb_v, gt_hbm.at[pl.ds(0, GCH)], swb)
            gath(k + 3, gefb_v, gtb_v, sgb)

    drain2(gefa_v, gef_hbm.at[pl.ds(0, GCH)], swa)
    drain2(gta_v, gt_hbm.at[pl.ds(0, GCH)], swa)
    drain2(gefb_v, gef_hbm.at[pl.ds(0, GCH)], swb)
    drain2(gtb_v, gt_hbm.at[pl.ds(0, GCH)], swb)


def _sc_regroup(row, ef, t16):
    out_type = (
        jax.ShapeDtypeStruct((GROWS, H), _f32),
        jax.ShapeDtypeStruct((GROWS, 128), _f32),
        jax.ShapeDtypeStruct((GROWS,), jnp.int32),
    )
    scratch = [
        pltpu.VMEM((SCHUNK,), jnp.int32),
        pltpu.VMEM((CAP,), jnp.int32),
        pltpu.VMEM((CAP,), jnp.int32),
        pltpu.VMEM((16 * CAP2,), jnp.int32),
        pltpu.VMEM((16 * CAP2,), jnp.int32),
        pltpu.VMEM((16,), jnp.int32),
        pltpu.VMEM((CAP2,), jnp.int32),
        pltpu.VMEM((CAP2,), jnp.int32),
        pltpu.VMEM((GCH, H), _f32),
        pltpu.VMEM((GCH, H), _f32),
        pltpu.VMEM((GCH, 128), _f32),
        pltpu.VMEM((GCH, 128), _f32),
        pltpu.VMEM_SHARED((16 * 16 * CAP2,), jnp.int32),
        pltpu.VMEM_SHARED((16 * 16 * CAP2,), jnp.int32),
        pltpu.VMEM_SHARED((256,), jnp.int32),
        pltpu.SMEM((257,), jnp.int32),
        pltpu.SemaphoreType.DMA,
        pltpu.SemaphoreType.DMA,
        pltpu.SemaphoreType.DMA,
        pltpu.SemaphoreType.DMA,
        pltpu.SemaphoreType.DMA,
    ]
    fn = pl.kernel(_regroup_body, out_type=out_type, mesh=_sc_mesh(),
                   scratch_types=scratch, compiler_params=_sc_params())
    return fn(row, ef, t16)


AGG_B = 512
AGG_K = CAP // AGG_B
AGG_R = 384


def _agg_kernel(lid_ref, gef_ref, gt_ref, nagg_ref, cagg_ref):
    k = pl.program_id(1)
    l2 = lid_ref[...].reshape(1, AGG_B)
    lb = jnp.broadcast_to(l2, (AGG_R, AGG_B))
    ohT = (lb == lax.broadcasted_iota(jnp.int32, (AGG_R, AGG_B), 0))
    ohT = ohT.astype(_bf16)
    c1 = jnp.dot(ohT, gef_ref[...].astype(_bf16),
                 preferred_element_type=_f32)
    c2 = jnp.dot(ohT, gt_ref[...].astype(_bf16),
                 preferred_element_type=_f32)

    @pl.when(k == 0)
    def _():
        nagg_ref[...] = jnp.zeros_like(nagg_ref)
        cagg_ref[...] = jnp.zeros_like(cagg_ref)

    nagg_ref[...] += c1
    cagg_ref[...] += c2


def _tc_agg(lid3, gef, gt):
    grid = (NW, AGG_K)
    return pl.pallas_call(
        _agg_kernel,
        grid=grid,
        in_specs=[
            pl.BlockSpec((1, 1, AGG_B), lambda w, k: (w * AGG_K + k, 0, 0)),
            pl.BlockSpec((AGG_B, H), lambda w, k: (w * AGG_K + k, 0)),
            pl.BlockSpec((AGG_B, 128), lambda w, k: (w * AGG_K + k, 0)),
        ],
        out_specs=[
            pl.BlockSpec((AGG_R, H), lambda w, k: (w, 0)),
            pl.BlockSpec((AGG_R, 128), lambda w, k: (w, 0)),
        ],
        out_shape=[
            jax.ShapeDtypeStruct((NW * AGG_R, H), _f32),
            jax.ShapeDtypeStruct((NW * AGG_R, 128), _f32),
        ],
    )(lid3, gef, gt)


def _silu(x):
    return x * jax.nn.sigmoid(x)


def _unpack_bf16(x):
    lo = lax.bitcast_convert_type(x << 16, _f32)
    hi = lax.bitcast_convert_type(x & jnp.int32(-65536), _f32)
    return lo, hi


def _edge_kernel(hr_ref, hc_ref, cdr_ref, ea_ref,
                 w1h_ref, w1er_ref, w2_ref, wc1_ref, wc2_ref, bias_ref,
                 ef_ref, t16_ref):
    lor, hir = _unpack_bf16(hr_ref[...])
    loc, hic = _unpack_bf16(hc_ref[...])
    hcat = jnp.concatenate([lor, hir, loc, hic], axis=1).astype(_bf16)
    cd = cdr_ref[...]
    radial = cd[:, 4:5]
    ea_ext = jnp.concatenate(
        [ea_ref[...], jnp.broadcast_to(radial, (radial.shape[0], 8))],
        axis=1).astype(_bf16)
    a = jnp.dot(hcat, w1h_ref[...], preferred_element_type=_f32)
    a += jnp.dot(ea_ext, w1er_ref[...], preferred_element_type=_f32)
    a += bias_ref[0:1, :]
    m = _silu(a).astype(_bf16)
    ef = jnp.dot(m, w2_ref[...], preferred_element_type=_f32)
    ef = _silu(ef + bias_ref[1:2, :])
    ef_ref[...] = ef
    cm = _silu(jnp.dot(ef.astype(_bf16), wc1_ref[...],
                       preferred_element_type=_f32) + bias_ref[2:3, :])
    scale = jnp.dot(cm.astype(_bf16), wc2_ref[...],
                    preferred_element_type=_f32)[:, 0:1]
    t8 = cd * scale
    t16_ref[...] = (jnp.concatenate(
        [t8, jnp.zeros((t8.shape[0], 120), _f32)], axis=1)
        + bias_ref[3:4, 0:128])


def _tc_edge(hr, hc, cdr, ea, w1h, w1er, w2, wc1, wc2p, bias):
    B = 1000
    grid = (E // B,)
    return pl.pallas_call(
        _edge_kernel,
        grid=grid,
        in_specs=[
            pl.BlockSpec((B, D // 2), lambda i: (i, 0)),
            pl.BlockSpec((B, D // 2), lambda i: (i, 0)),
            pl.BlockSpec((B, 8), lambda i: (i, 0)),
            pl.BlockSpec((B, DE), lambda i: (i, 0)),
            pl.BlockSpec((H, H), lambda i: (0, 0)),
            pl.BlockSpec((24, H), lambda i: (0, 0)),
            pl.BlockSpec((H, H), lambda i: (0, 0)),
            pl.BlockSpec((H, H), lambda i: (0, 0)),
            pl.BlockSpec((H, 128), lambda i: (0, 0)),
            pl.BlockSpec((8, H), lambda i: (0, 0)),
        ],
        out_specs=[
            pl.BlockSpec((B, H), lambda i: (i, 0)),
            pl.BlockSpec((B, 128), lambda i: (i, 0)),
        ],
        out_shape=[
            jax.ShapeDtypeStruct((E, H), _f32),
            jax.ShapeDtypeStruct((E, 128), _f32),
        ],
    )(hr, hc, cdr, ea, w1h, w1er, w2, wc1, wc2p, bias)


def _node_kernel(h_ref, nagg_ref, cagg_ref, cpad_ref,
                 w3a_ref, w3b_ref, w4_ref, bias_ref,
                 hout_ref, cout_ref):
    hb = h_ref[...].astype(_bf16)
    nb = nagg_ref[...].astype(_bf16)
    nhid = _silu(jnp.dot(hb, w3a_ref[...], preferred_element_type=_f32)
                 + jnp.dot(nb, w3b_ref[...], preferred_element_type=_f32)
                 + bias_ref[0:1, :])
    nout = jnp.dot(nhid.astype(_bf16), w4_ref[...],
                   preferred_element_type=_f32) + bias_ref[1:2, 0:D]
    hout_ref[...] = h_ref[...] + nout
    cagg = cagg_ref[...][:, 0:16]
    cnt = jnp.clip(cagg[:, 3:4], 1.0, None)
    cout_ref[...] = cpad_ref[...] + cagg / cnt


def _tc_node(h, nagg, cagg, cpad, w3a, w3b, w4, bias2):
    B = 1000
    grid = (N // B,)
    return pl.pallas_call(
        _node_kernel,
        grid=grid,
        in_specs=[
            pl.BlockSpec((B, D), lambda i: (i, 0)),
            pl.BlockSpec((B, H), lambda i: (i, 0)),
            pl.BlockSpec((B, 128), lambda i: (i, 0)),
            pl.BlockSpec((B, 16), lambda i: (i, 0)),
            pl.BlockSpec((D, H), lambda i: (0, 0)),
            pl.BlockSpec((H, H), lambda i: (0, 0)),
            pl.BlockSpec((H, D), lambda i: (0, 0)),
            pl.BlockSpec((8, H), lambda i: (0, 0)),
        ],
        out_specs=[
            pl.BlockSpec((B, D), lambda i: (i, 0)),
            pl.BlockSpec((B, 16), lambda i: (i, 0)),
        ],
        out_shape=[
            jax.ShapeDtypeStruct((N, D), _f32),
            jax.ShapeDtypeStruct((N, 16), _f32),
        ],
    )(h, nagg, cagg, cpad, w3a, w3b, w4, bias2)


def kernel(h, edge_index, coord, edge_attr,
           W1, b1, W2, b2, W3, b3, W4, b4, Wc1, bc1, Wc2):
    row = edge_index[0]
    col = edge_index[1]
    cpad = jnp.pad(coord, ((0, 0), (0, 13)))
    cflat = coord.reshape(3 * N)
    hpk = lax.bitcast_convert_type(
        h.astype(_bf16).reshape(N, D // 2, 2), jnp.int32)

    wr, wc = W1[:D], W1[D:2 * D]
    w1h = jnp.concatenate([wr[0::2], wr[1::2], wc[0::2], wc[1::2]],
                          axis=0).astype(_bf16)
    w1er = jnp.zeros((24, H), _f32).at[:DE].set(W1[2 * D + 1:]) \
        .at[DE].set(W1[2 * D]).astype(_bf16)
    bias = jnp.zeros((8, H), _f32).at[0].set(b1).at[1].set(b2) \
        .at[2].set(bc1).at[3, 3].set(1.0)
    wc2p = jnp.zeros((H, 128), _f32).at[:, 0].set(Wc2[:, 0]).astype(_bf16)
    w3a = W3[:D].astype(_bf16)
    w3b = W3[D:].astype(_bf16)
    w4 = W4.astype(_bf16)
    bias2 = jnp.zeros((8, H), _f32).at[0].set(b3).at[1, :D].set(b4)

    hr, hc, cdr = _sc_gather(hpk, cflat, row, col)
    ef, t16 = _tc_edge(hr, hc, cdr, edge_attr,
                       w1h, w1er, W2.astype(_bf16), Wc1.astype(_bf16),
                       wc2p, bias)
    gef, gt, lid = _sc_regroup(row, ef, t16)
    lid3 = lid.reshape(NW * AGG_K, 1, AGG_B)
    nagg_pad, cagg_pad = _tc_agg(lid3, gef, gt)
    nagg = nagg_pad.reshape(NW, AGG_R, H)[:, :NRANGE].reshape(NW * NRANGE, H)
    cagg = cagg_pad.reshape(NW, AGG_R, 128)[:, :NRANGE]
    cagg = cagg.reshape(NW * NRANGE, 128)
    h_out, cout16 = _tc_node(h, nagg[:N], cagg[:N], cpad,
                             w3a, w3b, w4, bias2)
    return h_out, cout16[:, :3], edge_attr


# speedup vs baseline: 1.1281x; 1.0698x over previous
import dataclasses

import jax
import jax.numpy as jnp
from jax import lax
from jax.experimental import pallas as pl
from jax.experimental.pallas import tpu as pltpu
from jax.experimental.pallas import tpu_sc as plsc

N = 10000
E = 160000
D = 256
H = 512
DE = 16

CHUNK = 128
NCHUNKS = E // CHUNK
NW = 32
NP = 2
NHALF = N // NP
ACC_ROWS = 5120
DUMMY = 5100
ZROWS = 320

_f32 = jnp.float32
_bf16 = jnp.bfloat16


def _sc_mesh():
    return plsc.VectorSubcoreMesh(core_axis_name="c", subcore_axis_name="s")


def _sc_params():
    cp = pltpu.CompilerParams()
    if "needs_layout_passes" in pltpu.CompilerParams.__dataclass_fields__:
        cp = dataclasses.replace(cp, needs_layout_passes=False)
    return cp


def _gather_body(hpk_hbm, cp4_hbm, row_hbm, col_hbm,
                 hr_out, hc_out, cdr_out,
                 idxr_v, idxc_v, hr_v, hc_v, cdr_v, coord_v,
                 s1, s2, s3):
    c = lax.axis_index("c")
    s = lax.axis_index("s")
    w = s * 2 + c

    pltpu.async_copy(cp4_hbm, coord_v, s3).wait()
    zero16 = jnp.zeros((16,), _f32)
    for t in (5, 6, 7):
        tcol = jnp.full((16,), t, jnp.int32)

        @pl.loop(0, CHUNK // 16)
        def _(j):
            ridx = j * 16 + lax.iota(jnp.int32, 16)
            plsc.store_scatter(cdr_v, [ridx, tcol], zero16)

    @pl.loop(0, 40)
    def _(k):
        chunk = w + NW * k

        @pl.when(chunk < NCHUNKS)
        def _():
            base = chunk * CHUNK
            pltpu.sync_copy(row_hbm.at[pl.ds(base, CHUNK)], idxr_v)
            pltpu.sync_copy(col_hbm.at[pl.ds(base, CHUNK)], idxc_v)
            d1 = pltpu.async_copy(hpk_hbm.at[idxr_v], hr_v, s1)
            d2 = pltpu.async_copy(hpk_hbm.at[idxc_v], hc_v, s2)

            @pl.loop(0, CHUNK // 16)
            def _(j):
                r16 = idxr_v[pl.ds(j * 16, 16)] * 3
                c16 = idxc_v[pl.ds(j * 16, 16)] * 3
                ridx = j * 16 + lax.iota(jnp.int32, 16)
                rad = jnp.zeros((16,), _f32)
                for t in range(3):
                    tcol = jnp.full((16,), t, jnp.int32)
                    a = plsc.load_gather(coord_v, [r16 + t])
                    b = plsc.load_gather(coord_v, [c16 + t])
                    dv = a - b
                    plsc.store_scatter(cdr_v, [ridx, tcol], dv)
                    rad = rad + dv * dv
                plsc.store_scatter(cdr_v, [ridx, jnp.full((16,), 4, jnp.int32)],
                                   rad)

            d1.wait()
            d2.wait()
            pltpu.sync_copy(hr_v, hr_out.at[pl.ds(base, CHUNK)])
            pltpu.sync_copy(hc_v, hc_out.at[pl.ds(base, CHUNK)])
            pltpu.sync_copy(cdr_v, cdr_out.at[pl.ds(base, CHUNK)])


def _sc_gather(hpk, cpad4, row, col):
    out_type = (
        jax.ShapeDtypeStruct((E, D // 2), jnp.int32),
        jax.ShapeDtypeStruct((E, D // 2), jnp.int32),
        jax.ShapeDtypeStruct((E, 8), _f32),
    )
    scratch = [
        pltpu.VMEM((CHUNK,), jnp.int32),
        pltpu.VMEM((CHUNK,), jnp.int32),
        pltpu.VMEM((CHUNK, D // 2), jnp.int32),
        pltpu.VMEM((CHUNK, D // 2), jnp.int32),
        pltpu.VMEM((CHUNK, 8), _f32),
        pltpu.VMEM((3 * N,), _f32),
        pltpu.SemaphoreType.DMA,
        pltpu.SemaphoreType.DMA,
        pltpu.SemaphoreType.DMA,
    ]
    fn = pl.kernel(_gather_body, out_type=out_type, mesh=_sc_mesh(),
                   scratch_types=scratch, compiler_params=_sc_params())
    return fn(hpk, cpad4, row, col)


NRANGE = 320
CAP = 5632
GROWS = NW * CAP
SCHUNK = 2000
LDUMMY = NRANGE


GCH = 32
NGCH = CAP // GCH
CAP2 = 832
SLICE = E // 16


def _regroup_body(row_hbm, ef_hbm, t16_hbm,
                  gef_hbm, gt_hbm, lid_hbm,
                  rbuf_v, ids_v, lid_v, binid_v, binlid_v, off_v,
                  mids_v, mlids_v, gefa_v, gefb_v, gta_v, gtb_v,
                  stag_ids, stag_lids, stag_cnt,
                  cnt_s, sd, sga, sgb, swa, swb):
    c = lax.axis_index("c")
    s = lax.axis_index("s")
    w = c * 16 + s
    iota16 = lax.iota(jnp.int32, 16)
    ones16 = jnp.ones((16,), jnp.int32)

    zid = jnp.zeros((16,), jnp.int32)
    ldm = jnp.full((16,), LDUMMY, jnp.int32)

    @pl.loop(0, CAP // 16)
    def _(i):
        ids_v[pl.ds(i * 16, 16)] = zid
        lid_v[pl.ds(i * 16, 16)] = ldm

    off_v[pl.ds(0, 16)] = jnp.zeros((16,), jnp.int32)

    @pl.loop(0, SLICE // SCHUNK)
    def _(k):
        pltpu.sync_copy(row_hbm.at[pl.ds(s * SLICE + k * SCHUNK, SCHUNK)],
                        rbuf_v)

        @pl.loop(0, SCHUNK // 16)
        def _(g):
            r16 = rbuf_v[pl.ds(g * 16, 16)]
            rel = r16 - c * (16 * NRANGE)
            m = (rel >= 0) & (rel < 16 * NRANGE)
            b16 = jnp.clip(rel // NRANGE, 0, 15)
            cnt1, lastm = plsc.scan_count(b16, mask=m)
            basev = plsc.load_gather(off_v, [b16])
            addr = b16 * CAP2 + basev + (cnt1 - 1)
            eid = (s * SLICE + k * SCHUNK + g * 16) + iota16
            plsc.store_scatter(binid_v, [addr], eid, mask=m)
            plsc.store_scatter(binlid_v, [addr], rel - b16 * NRANGE, mask=m)
            plsc.addupdate_scatter(off_v, [b16], cnt1, mask=m & lastm)

    @pl.loop(0, 16)
    def _(r):
        pltpu.sync_copy(binid_v.at[pl.ds(r * CAP2, CAP2)],
                        stag_ids.at[pl.ds((s * 16 + r) * CAP2, CAP2)])
        pltpu.sync_copy(binlid_v.at[pl.ds(r * CAP2, CAP2)],
                        stag_lids.at[pl.ds((s * 16 + r) * CAP2, CAP2)])

    pltpu.sync_copy(off_v, stag_cnt.at[pl.ds(s * 16, 16)])
    plsc.subcore_barrier()

    pltpu.async_copy(stag_cnt, cnt_s.at[pl.ds(0, 256)], sd).wait()
    cum_init = 0
    cnt_s[256] = cum_init
    for t in range(16):
        pltpu.sync_copy(stag_ids.at[pl.ds((t * 16 + s) * CAP2, CAP2)], mids_v)
        pltpu.sync_copy(stag_lids.at[pl.ds((t * 16 + s) * CAP2, CAP2)],
                        mlids_v)
        cnt_t = cnt_s[t * 16 + s]
        cum = cnt_s[256]

        @pl.loop(0, CAP2 // 16)
        def _(j):
            o = j * 16

            @pl.when(o < cnt_t)
            def _():
                mk = (o + iota16) < cnt_t
                plsc.store_compressed(ids_v.at[pl.ds(cum + o, 16)],
                                      mids_v[pl.ds(o, 16)], mask=mk)
                plsc.store_compressed(lid_v.at[pl.ds(cum + o, 16)],
                                      mlids_v[pl.ds(o, 16)], mask=mk)

        cnt_s[256] = cum + cnt_t

    pltpu.sync_copy(lid_v, lid_hbm.at[pl.ds(w * CAP, CAP)])

    def gath(k, buf_ef, buf_t, sem):
        iv = ids_v.at[pl.ds(k * GCH, GCH)]
        pltpu.async_copy(ef_hbm.at[iv], buf_ef, sem)
        pltpu.async_copy(t16_hbm.at[iv], buf_t, sem)

    def wr(k, buf_ef, buf_t, sem):
        base = w * CAP + k * GCH
        pltpu.async_copy(buf_ef, gef_hbm.at[pl.ds(base, GCH)], sem)
        pltpu.async_copy(buf_t, gt_hbm.at[pl.ds(base, GCH)], sem)

    def drain2(src, dst, sem):
        pltpu.make_async_copy(src, dst, sem).wait()

    gath(0, gefa_v, gta_v, sga)
    gath(1, gefb_v, gtb_v, sgb)

    @pl.loop(0, NGCH // 2)
    def _(i):
        k = 2 * i
        drain2(ef_hbm.at[pl.ds(0, GCH)], gefa_v, sga)
        drain2(t16_hbm.at[pl.ds(0, GCH)], gta_v, sga)
        wr(k, gefa_v, gta_v, swa)
        drain2(ef_hbm.at[pl.ds(0, GCH)], gefb_v, sgb)
        drain2(t16_hbm.at[pl.ds(0, GCH)], gtb_v, sgb)
        wr(k + 1, gefb_v, gtb_v, swb)

        @pl.when(k + 2 < NGCH)
        def _():
            drain2(gefa_v, gef_hbm.at[pl.ds(0, GCH)], swa)
            drain2(gta_v, gt_hbm.at[pl.ds(0, GCH)], swa)
            gath(k + 2, gefa_v, gta_v, sga)
            drain2(gefb_v, gef_hbm.at[pl.ds(0, GCH)], swb)
            drain2(gtb_v, gt_hbm.at[pl.ds(0, GCH)], swb)
            gath(k + 3, gefb_v, gtb_v, sgb)

    drain2(gefa_v, gef_hbm.at[pl.ds(0, GCH)], swa)
    drain2(gta_v, gt_hbm.at[pl.ds(0, GCH)], swa)
    drain2(gefb_v, gef_hbm.at[pl.ds(0, GCH)], swb)
    drain2(gtb_v, gt_hbm.at[pl.ds(0, GCH)], swb)


def _sc_regroup(row, ef, t16):
    out_type = (
        jax.ShapeDtypeStruct((GROWS, 256), jnp.int32),
        jax.ShapeDtypeStruct((GROWS, 128), _f32),
        jax.ShapeDtypeStruct((GROWS,), jnp.int32),
    )
    scratch = [
        pltpu.VMEM((SCHUNK,), jnp.int32),
        pltpu.VMEM((CAP,), jnp.int32),
        pltpu.VMEM((CAP,), jnp.int32),
        pltpu.VMEM((16 * CAP2,), jnp.int32),
        pltpu.VMEM((16 * CAP2,), jnp.int32),
        pltpu.VMEM((16,), jnp.int32),
        pltpu.VMEM((CAP2,), jnp.int32),
        pltpu.VMEM((CAP2,), jnp.int32),
        pltpu.VMEM((GCH, 256), jnp.int32),
        pltpu.VMEM((GCH, 256), jnp.int32),
        pltpu.VMEM((GCH, 128), _f32),
        pltpu.VMEM((GCH, 128), _f32),
        pltpu.VMEM_SHARED((16 * 16 * CAP2,), jnp.int32),
        pltpu.VMEM_SHARED((16 * 16 * CAP2,), jnp.int32),
        pltpu.VMEM_SHARED((256,), jnp.int32),
        pltpu.SMEM((257,), jnp.int32),
        pltpu.SemaphoreType.DMA,
        pltpu.SemaphoreType.DMA,
        pltpu.SemaphoreType.DMA,
        pltpu.SemaphoreType.DMA,
        pltpu.SemaphoreType.DMA,
    ]
    fn = pl.kernel(_regroup_body, out_type=out_type, mesh=_sc_mesh(),
                   scratch_types=scratch, compiler_params=_sc_params())
    return fn(row, ef, t16)


AGG_B = 512
AGG_K = CAP // AGG_B
AGG_R = 384


def _agg_kernel(lid_ref, gef_ref, gt_ref, nagg_ref, cagg_ref):
    k = pl.program_id(1)
    l2 = lid_ref[...].reshape(1, AGG_B)
    lb = jnp.broadcast_to(l2, (AGG_R, AGG_B))
    ohT = (lb == lax.broadcasted_iota(jnp.int32, (AGG_R, AGG_B), 0))
    ohT = ohT.astype(_bf16)
    glo, ghi = _unpack_bf16(gef_ref[...])
    gef = jnp.concatenate([glo, ghi], axis=1).astype(_bf16)
    c1 = jnp.dot(ohT, gef, preferred_element_type=_f32)
    c2 = jnp.dot(ohT, gt_ref[...].astype(_bf16),
                 preferred_element_type=_f32)

    @pl.when(k == 0)
    def _():
        nagg_ref[...] = jnp.zeros_like(nagg_ref)
        cagg_ref[...] = jnp.zeros_like(cagg_ref)

    nagg_ref[...] += c1
    cagg_ref[...] += c2


def _tc_agg(lid3, gef, gt):
    grid = (NW, AGG_K)
    return pl.pallas_call(
        _agg_kernel,
        grid=grid,
        in_specs=[
            pl.BlockSpec((1, 1, AGG_B), lambda w, k: (w * AGG_K + k, 0, 0)),
            pl.BlockSpec((AGG_B, 256), lambda w, k: (w * AGG_K + k, 0)),
            pl.BlockSpec((AGG_B, 128), lambda w, k: (w * AGG_K + k, 0)),
        ],
        out_specs=[
            pl.BlockSpec((AGG_R, H), lambda w, k: (w, 0)),
            pl.BlockSpec((AGG_R, 128), lambda w, k: (w, 0)),
        ],
        out_shape=[
            jax.ShapeDtypeStruct((NW * AGG_R, H), _f32),
            jax.ShapeDtypeStruct((NW * AGG_R, 128), _f32),
        ],
    )(lid3, gef, gt)


def _silu(x):
    return x * jax.nn.sigmoid(x)


def _unpack_bf16(x):
    lo = lax.bitcast_convert_type(x << 16, _f32)
    hi = lax.bitcast_convert_type(x & jnp.int32(-65536), _f32)
    return lo, hi


def _edge_kernel(hr_ref, hc_ref, cdr_ref, ea_ref,
                 w1h_ref, w1er_ref, w2_ref, wc1_ref, wc2_ref, bias_ref,
                 ef_ref, t16_ref):
    lor, hir = _unpack_bf16(hr_ref[...])
    loc, hic = _unpack_bf16(hc_ref[...])
    hcat = jnp.concatenate([lor, hir, loc, hic], axis=1).astype(_bf16)
    cd = cdr_ref[...]
    radial = cd[:, 4:5]
    ea_ext = jnp.concatenate(
        [ea_ref[...], jnp.broadcast_to(radial, (radial.shape[0], 8))],
        axis=1).astype(_bf16)
    a = jnp.dot(hcat, w1h_ref[...], preferred_element_type=_f32)
    a += jnp.dot(ea_ext, w1er_ref[...], preferred_element_type=_f32)
    a += bias_ref[0:1, :]
    m = _silu(a).astype(_bf16)
    ef = jnp.dot(m, w2_ref[...], preferred_element_type=_f32)
    ef = _silu(ef + bias_ref[1:2, :])
    efb = ef.astype(_bf16)
    av = lax.bitcast_convert_type(efb[:, :256], jnp.uint16).astype(jnp.uint32)
    bv = lax.bitcast_convert_type(efb[:, 256:], jnp.uint16).astype(jnp.uint32)
    ef_ref[...] = lax.bitcast_convert_type(av | (bv << 16), jnp.int32)
    cm = _silu(jnp.dot(efb, wc1_ref[...],
                       preferred_element_type=_f32) + bias_ref[2:3, :])
    scale = jnp.dot(cm.astype(_bf16), wc2_ref[...],
                    preferred_element_type=_f32)[:, 0:1]
    t8 = cd * scale
    t16_ref[...] = (jnp.concatenate(
        [t8, jnp.zeros((t8.shape[0], 120), _f32)], axis=1)
        + bias_ref[3:4, 0:128])


def _tc_edge(hr, hc, cdr, ea, w1h, w1er, w2, wc1, wc2p, bias):
    B = 1000
    grid = (E // B,)
    return pl.pallas_call(
        _edge_kernel,
        grid=grid,
        in_specs=[
            pl.BlockSpec((B, D // 2), lambda i: (i, 0)),
            pl.BlockSpec((B, D // 2), lambda i: (i, 0)),
            pl.BlockSpec((B, 8), lambda i: (i, 0)),
            pl.BlockSpec((B, DE), lambda i: (i, 0)),
            pl.BlockSpec((H, H), lambda i: (0, 0)),
            pl.BlockSpec((24, H), lambda i: (0, 0)),
            pl.BlockSpec((H, H), lambda i: (0, 0)),
            pl.BlockSpec((H, H), lambda i: (0, 0)),
            pl.BlockSpec((H, 128), lambda i: (0, 0)),
            pl.BlockSpec((8, H), lambda i: (0, 0)),
        ],
        out_specs=[
            pl.BlockSpec((B, 256), lambda i: (i, 0)),
            pl.BlockSpec((B, 128), lambda i: (i, 0)),
        ],
        out_shape=[
            jax.ShapeDtypeStruct((E, 256), jnp.int32),
            jax.ShapeDtypeStruct((E, 128), _f32),
        ],
    )(hr, hc, cdr, ea, w1h, w1er, w2, wc1, wc2p, bias)


def _node_kernel(h_ref, nagg_ref, cagg_ref, cpad_ref,
                 w3a_ref, w3b_ref, w4_ref, bias_ref,
                 hout_ref, cout_ref):
    hb = h_ref[...].astype(_bf16)
    nb = nagg_ref[...].astype(_bf16)
    nhid = _silu(jnp.dot(hb, w3a_ref[...], preferred_element_type=_f32)
                 + jnp.dot(nb, w3b_ref[...], preferred_element_type=_f32)
                 + bias_ref[0:1, :])
    nout = jnp.dot(nhid.astype(_bf16), w4_ref[...],
                   preferred_element_type=_f32) + bias_ref[1:2, 0:D]
    hout_ref[...] = h_ref[...] + nout
    cagg = cagg_ref[...][:, 0:16]
    cnt = jnp.clip(cagg[:, 3:4], 1.0, None)
    cout_ref[...] = cpad_ref[...] + cagg / cnt


def _tc_node(h, nagg, cagg, cpad, w3a, w3b, w4, bias2):
    B = 1000
    grid = (N // B,)
    return pl.pallas_call(
        _node_kernel,
        grid=grid,
        in_specs=[
            pl.BlockSpec((B, D), lambda i: (i, 0)),
            pl.BlockSpec((B, H), lambda i: (i, 0)),
            pl.BlockSpec((B, 128), lambda i: (i, 0)),
            pl.BlockSpec((B, 16), lambda i: (i, 0)),
            pl.BlockSpec((D, H), lambda i: (0, 0)),
            pl.BlockSpec((H, H), lambda i: (0, 0)),
            pl.BlockSpec((H, D), lambda i: (0, 0)),
            pl.BlockSpec((8, H), lambda i: (0, 0)),
        ],
        out_specs=[
            pl.BlockSpec((B, D), lambda i: (i, 0)),
            pl.BlockSpec((B, 16), lambda i: (i, 0)),
        ],
        out_shape=[
            jax.ShapeDtypeStruct((N, D), _f32),
            jax.ShapeDtypeStruct((N, 16), _f32),
        ],
    )(h, nagg, cagg, cpad, w3a, w3b, w4, bias2)


def kernel(h, edge_index, coord, edge_attr,
           W1, b1, W2, b2, W3, b3, W4, b4, Wc1, bc1, Wc2):
    row = edge_index[0]
    col = edge_index[1]
    cpad = jnp.pad(coord, ((0, 0), (0, 13)))
    cflat = coord.reshape(3 * N)
    hpk = lax.bitcast_convert_type(
        h.astype(_bf16).reshape(N, D // 2, 2), jnp.int32)

    wr, wc = W1[:D], W1[D:2 * D]
    w1h = jnp.concatenate([wr[0::2], wr[1::2], wc[0::2], wc[1::2]],
                          axis=0).astype(_bf16)
    w1er = jnp.zeros((24, H), _f32).at[:DE].set(W1[2 * D + 1:]) \
        .at[DE].set(W1[2 * D]).astype(_bf16)
    bias = jnp.zeros((8, H), _f32).at[0].set(b1).at[1].set(b2) \
        .at[2].set(bc1).at[3, 3].set(1.0)
    wc2p = jnp.zeros((H, 128), _f32).at[:, 0].set(Wc2[:, 0]).astype(_bf16)
    w3a = W3[:D].astype(_bf16)
    w3b = W3[D:].astype(_bf16)
    w4 = W4.astype(_bf16)
    bias2 = jnp.zeros((8, H), _f32).at[0].set(b3).at[1, :D].set(b4)

    hr, hc, cdr = _sc_gather(hpk, cflat, row, col)
    ef, t16 = _tc_edge(hr, hc, cdr, edge_attr,
                       w1h, w1er, W2.astype(_bf16), Wc1.astype(_bf16),
                       wc2p, bias)
    gef, gt, lid = _sc_regroup(row, ef, t16)
    lid3 = lid.reshape(NW * AGG_K, 1, AGG_B)
    nagg_pad, cagg_pad = _tc_agg(lid3, gef, gt)
    nagg = nagg_pad.reshape(NW, AGG_R, H)[:, :NRANGE].reshape(NW * NRANGE, H)
    cagg = cagg_pad.reshape(NW, AGG_R, 128)[:, :NRANGE]
    cagg = cagg.reshape(NW * NRANGE, 128)
    h_out, cout16 = _tc_node(h, nagg[:N], cagg[:N], cpad,
                             w3a, w3b, w4, bias2)
    return h_out, cout16[:, :3], edge_attr


# speedup vs baseline: 1.1475x; 1.0172x over previous
import dataclasses

import jax
import jax.numpy as jnp
from jax import lax
from jax.experimental import pallas as pl
from jax.experimental.pallas import tpu as pltpu
from jax.experimental.pallas import tpu_sc as plsc

N = 10000
E = 160000
D = 256
H = 512
DE = 16

CHUNK = 128
NCHUNKS = E // CHUNK
NW = 32
NP = 2
NHALF = N // NP
ACC_ROWS = 5120
DUMMY = 5100
ZROWS = 320

_f32 = jnp.float32
_bf16 = jnp.bfloat16


def _sc_mesh():
    return plsc.VectorSubcoreMesh(core_axis_name="c", subcore_axis_name="s")


def _sc_params():
    cp = pltpu.CompilerParams()
    if "needs_layout_passes" in pltpu.CompilerParams.__dataclass_fields__:
        cp = dataclasses.replace(cp, needs_layout_passes=False)
    return cp


def _gather_body(hpk_hbm, cp4_hbm, row_hbm, col_hbm,
                 hr_out, hc_out, cdr_out,
                 idxr_v, idxc_v, hr_v, hc_v, cdr_v, coord_v,
                 s1, s2, s3):
    c = lax.axis_index("c")
    s = lax.axis_index("s")
    w = s * 2 + c

    pltpu.async_copy(cp4_hbm, coord_v, s3).wait()
    zero16 = jnp.zeros((16,), _f32)
    for t in (5, 6, 7):
        tcol = jnp.full((16,), t, jnp.int32)

        @pl.loop(0, CHUNK // 16)
        def _(j):
            ridx = j * 16 + lax.iota(jnp.int32, 16)
            plsc.store_scatter(cdr_v, [ridx, tcol], zero16)

    @pl.loop(0, 40)
    def _(k):
        chunk = w + NW * k

        @pl.when(chunk < NCHUNKS)
        def _():
            base = chunk * CHUNK
            pltpu.sync_copy(row_hbm.at[pl.ds(base, CHUNK)], idxr_v)
            pltpu.sync_copy(col_hbm.at[pl.ds(base, CHUNK)], idxc_v)
            d1 = pltpu.async_copy(hpk_hbm.at[idxr_v], hr_v, s1)
            d2 = pltpu.async_copy(hpk_hbm.at[idxc_v], hc_v, s2)

            @pl.loop(0, CHUNK // 16)
            def _(j):
                r16 = idxr_v[pl.ds(j * 16, 16)] * 3
                c16 = idxc_v[pl.ds(j * 16, 16)] * 3
                ridx = j * 16 + lax.iota(jnp.int32, 16)
                rad = jnp.zeros((16,), _f32)
                for t in range(3):
                    tcol = jnp.full((16,), t, jnp.int32)
                    a = plsc.load_gather(coord_v, [r16 + t])
                    b = plsc.load_gather(coord_v, [c16 + t])
                    dv = a - b
                    plsc.store_scatter(cdr_v, [ridx, tcol], dv)
                    rad = rad + dv * dv
                plsc.store_scatter(cdr_v, [ridx, jnp.full((16,), 4, jnp.int32)],
                                   rad)

            d1.wait()
            d2.wait()
            pltpu.sync_copy(hr_v, hr_out.at[pl.ds(base, CHUNK)])
            pltpu.sync_copy(hc_v, hc_out.at[pl.ds(base, CHUNK)])
            pltpu.sync_copy(cdr_v, cdr_out.at[pl.ds(base, CHUNK)])


def _sc_gather(hpk, cpad4, row, col):
    out_type = (
        jax.ShapeDtypeStruct((E, D // 2), jnp.int32),
        jax.ShapeDtypeStruct((E, D // 2), jnp.int32),
        jax.ShapeDtypeStruct((E, 8), _f32),
    )
    scratch = [
        pltpu.VMEM((CHUNK,), jnp.int32),
        pltpu.VMEM((CHUNK,), jnp.int32),
        pltpu.VMEM((CHUNK, D // 2), jnp.int32),
        pltpu.VMEM((CHUNK, D // 2), jnp.int32),
        pltpu.VMEM((CHUNK, 8), _f32),
        pltpu.VMEM((3 * N,), _f32),
        pltpu.SemaphoreType.DMA,
        pltpu.SemaphoreType.DMA,
        pltpu.SemaphoreType.DMA,
    ]
    fn = pl.kernel(_gather_body, out_type=out_type, mesh=_sc_mesh(),
                   scratch_types=scratch, compiler_params=_sc_params())
    return fn(hpk, cpad4, row, col)


NRANGE = 320
CAP = 5632
GROWS = NW * CAP
SCHUNK = 2000
LDUMMY = NRANGE


GCH = 32
NGCH = CAP // GCH
CAP2 = 832
SLICE = E // 16


def _regroup_body(row_hbm, ef_hbm, t16_hbm,
                  gef_hbm, gt_hbm, lid_hbm,
                  rbuf_v, ids_v, lid_v, binid_v, binlid_v, off_v,
                  mids_v, mlids_v, gefa_v, gefb_v, gta_v, gtb_v,
                  stag_ids, stag_lids, stag_cnt,
                  cnt_s, sd, sga, sgb, swa, swb):
    c = lax.axis_index("c")
    s = lax.axis_index("s")
    w = c * 16 + s
    iota16 = lax.iota(jnp.int32, 16)
    ones16 = jnp.ones((16,), jnp.int32)

    zid = jnp.zeros((16,), jnp.int32)
    ldm = jnp.full((16,), LDUMMY, jnp.int32)

    @pl.loop(0, CAP // 16)
    def _(i):
        ids_v[pl.ds(i * 16, 16)] = zid
        lid_v[pl.ds(i * 16, 16)] = ldm

    off_v[pl.ds(0, 16)] = jnp.zeros((16,), jnp.int32)

    @pl.loop(0, SLICE // SCHUNK)
    def _(k):
        pltpu.sync_copy(row_hbm.at[pl.ds(s * SLICE + k * SCHUNK, SCHUNK)],
                        rbuf_v)

        @pl.loop(0, SCHUNK // 16)
        def _(g):
            r16 = rbuf_v[pl.ds(g * 16, 16)]
            rel = r16 - c * (16 * NRANGE)
            m = (rel >= 0) & (rel < 16 * NRANGE)
            b16 = jnp.clip(rel // NRANGE, 0, 15)
            cnt1, lastm = plsc.scan_count(b16, mask=m)
            basev = plsc.load_gather(off_v, [b16])
            addr = b16 * CAP2 + basev + (cnt1 - 1)
            eid = (s * SLICE + k * SCHUNK + g * 16) + iota16
            plsc.store_scatter(binid_v, [addr], eid, mask=m)
            plsc.store_scatter(binlid_v, [addr], rel - b16 * NRANGE, mask=m)
            plsc.addupdate_scatter(off_v, [b16], cnt1, mask=m & lastm)

    @pl.loop(0, 16)
    def _(r):
        pltpu.sync_copy(binid_v.at[pl.ds(r * CAP2, CAP2)],
                        stag_ids.at[pl.ds((s * 16 + r) * CAP2, CAP2)])
        pltpu.sync_copy(binlid_v.at[pl.ds(r * CAP2, CAP2)],
                        stag_lids.at[pl.ds((s * 16 + r) * CAP2, CAP2)])

    pltpu.sync_copy(off_v, stag_cnt.at[pl.ds(s * 16, 16)])
    plsc.subcore_barrier()

    pltpu.async_copy(stag_cnt, cnt_s.at[pl.ds(0, 256)], sd).wait()
    cum_init = 0
    cnt_s[256] = cum_init
    for t in range(16):
        pltpu.sync_copy(stag_ids.at[pl.ds((t * 16 + s) * CAP2, CAP2)], mids_v)
        pltpu.sync_copy(stag_lids.at[pl.ds((t * 16 + s) * CAP2, CAP2)],
                        mlids_v)
        cnt_t = cnt_s[t * 16 + s]
        cum = cnt_s[256]

        @pl.loop(0, CAP2 // 16)
        def _(j):
            o = j * 16

            @pl.when(o < cnt_t)
            def _():
                mk = (o + iota16) < cnt_t
                plsc.store_compressed(ids_v.at[pl.ds(cum + o, 16)],
                                      mids_v[pl.ds(o, 16)], mask=mk)
                plsc.store_compressed(lid_v.at[pl.ds(cum + o, 16)],
                                      mlids_v[pl.ds(o, 16)], mask=mk)

        cnt_s[256] = cum + cnt_t

    pltpu.sync_copy(lid_v, lid_hbm.at[pl.ds(w * CAP, CAP)])

    def gath(k, buf_ef, buf_t, sem):
        iv = ids_v.at[pl.ds(k * GCH, GCH)]
        pltpu.async_copy(ef_hbm.at[iv], buf_ef, sem)
        pltpu.async_copy(t16_hbm.at[iv], buf_t, sem)

    def wr(k, buf_ef, buf_t, sem):
        base = w * CAP + k * GCH
        pltpu.async_copy(buf_ef, gef_hbm.at[pl.ds(base, GCH)], sem)
        pltpu.async_copy(buf_t, gt_hbm.at[pl.ds(base, GCH)], sem)

    def drain2(src, dst, sem):
        pltpu.make_async_copy(src, dst, sem).wait()

    gath(0, gefa_v, gta_v, sga)
    gath(1, gefb_v, gtb_v, sgb)

    @pl.loop(0, NGCH // 2)
    def _(i):
        k = 2 * i
        drain2(ef_hbm.at[pl.ds(0, GCH)], gefa_v, sga)
        drain2(t16_hbm.at[pl.ds(0, GCH)], gta_v, sga)
        wr(k, gefa_v, gta_v, swa)
        drain2(ef_hbm.at[pl.ds(0, GCH)], gefb_v, sgb)
        drain2(t16_hbm.at[pl.ds(0, GCH)], gtb_v, sgb)
        wr(k + 1, gefb_v, gtb_v, swb)

        @pl.when(k + 2 < NGCH)
        def _():
            drain2(gefa_v, gef_hbm.at[pl.ds(0, GCH)], swa)
            drain2(gta_v, gt_hbm.at[pl.ds(0, GCH)], swa)
            gath(k + 2, gefa_v, gta_v, sga)
            drain2(gefb_v, gef_hbm.at[pl.ds(0, GCH)], swb)
            drain2(gtb_v, gt_hbm.at[pl.ds(0, GCH)], swb)
            gath(k + 3, gefb_v, gtb_v, sgb)

    drain2(gefa_v, gef_hbm.at[pl.ds(0, GCH)], swa)
    drain2(gta_v, gt_hbm.at[pl.ds(0, GCH)], swa)
    drain2(gefb_v, gef_hbm.at[pl.ds(0, GCH)], swb)
    drain2(gtb_v, gt_hbm.at[pl.ds(0, GCH)], swb)


def _sc_regroup(row, ef, t16):
    out_type = (
        jax.ShapeDtypeStruct((GROWS, 256), jnp.int32),
        jax.ShapeDtypeStruct((GROWS, 128), _f32),
        jax.ShapeDtypeStruct((GROWS,), jnp.int32),
    )
    scratch = [
        pltpu.VMEM((SCHUNK,), jnp.int32),
        pltpu.VMEM((CAP,), jnp.int32),
        pltpu.VMEM((CAP,), jnp.int32),
        pltpu.VMEM((16 * CAP2,), jnp.int32),
        pltpu.VMEM((16 * CAP2,), jnp.int32),
        pltpu.VMEM((16,), jnp.int32),
        pltpu.VMEM((CAP2,), jnp.int32),
        pltpu.VMEM((CAP2,), jnp.int32),
        pltpu.VMEM((GCH, 256), jnp.int32),
        pltpu.VMEM((GCH, 256), jnp.int32),
        pltpu.VMEM((GCH, 128), _f32),
        pltpu.VMEM((GCH, 128), _f32),
        pltpu.VMEM_SHARED((16 * 16 * CAP2,), jnp.int32),
        pltpu.VMEM_SHARED((16 * 16 * CAP2,), jnp.int32),
        pltpu.VMEM_SHARED((256,), jnp.int32),
        pltpu.SMEM((257,), jnp.int32),
        pltpu.SemaphoreType.DMA,
        pltpu.SemaphoreType.DMA,
        pltpu.SemaphoreType.DMA,
        pltpu.SemaphoreType.DMA,
        pltpu.SemaphoreType.DMA,
    ]
    fn = pl.kernel(_regroup_body, out_type=out_type, mesh=_sc_mesh(),
                   scratch_types=scratch, compiler_params=_sc_params())
    return fn(row, ef, t16)


AGG_B = 512
AGG_K = CAP // AGG_B
AGG_R = 384


def _agg_kernel(lid_ref, gef_ref, gt_ref, nagg_ref, cagg_ref):
    k = pl.program_id(1)
    l2 = lid_ref[...].reshape(1, AGG_B)
    lb = jnp.broadcast_to(l2, (AGG_R, AGG_B))
    ohT = (lb == lax.broadcasted_iota(jnp.int32, (AGG_R, AGG_B), 0))
    ohT = ohT.astype(_bf16)
    glo, ghi = _unpack_bf16(gef_ref[...])
    gef = jnp.concatenate([glo, ghi], axis=1).astype(_bf16)
    c1 = jnp.dot(ohT, gef, preferred_element_type=_f32)
    c2 = jnp.dot(ohT, gt_ref[...].astype(_bf16),
                 preferred_element_type=_f32)

    @pl.when(k == 0)
    def _():
        nagg_ref[...] = jnp.zeros_like(nagg_ref)
        cagg_ref[...] = jnp.zeros_like(cagg_ref)

    nagg_ref[...] += c1
    cagg_ref[...] += c2


def _tc_agg(lid3, gef, gt):
    grid = (NW, AGG_K)
    return pl.pallas_call(
        _agg_kernel,
        grid=grid,
        in_specs=[
            pl.BlockSpec((1, 1, AGG_B), lambda w, k: (w * AGG_K + k, 0, 0)),
            pl.BlockSpec((AGG_B, 256), lambda w, k: (w * AGG_K + k, 0)),
            pl.BlockSpec((AGG_B, 128), lambda w, k: (w * AGG_K + k, 0)),
        ],
        out_specs=[
            pl.BlockSpec((AGG_R, H), lambda w, k: (w, 0)),
            pl.BlockSpec((AGG_R, 128), lambda w, k: (w, 0)),
        ],
        out_shape=[
            jax.ShapeDtypeStruct((NW * AGG_R, H), _f32),
            jax.ShapeDtypeStruct((NW * AGG_R, 128), _f32),
        ],
    )(lid3, gef, gt)


def _silu(x):
    return x * jax.nn.sigmoid(x)


def _silu_bf(x):
    xb = x.astype(_bf16)
    return xb * jax.nn.sigmoid(xb)


def _unpack_bf16(x):
    lo = lax.bitcast_convert_type(x << 16, _f32)
    hi = lax.bitcast_convert_type(x & jnp.int32(-65536), _f32)
    return lo, hi


def _edge_kernel(hr_ref, hc_ref, cdr_ref, ea_ref,
                 w1h_ref, w1er_ref, w2_ref, wc1_ref, wc2_ref, bias_ref,
                 ef_ref, t16_ref):
    lor, hir = _unpack_bf16(hr_ref[...])
    loc, hic = _unpack_bf16(hc_ref[...])
    hcat = jnp.concatenate([lor, hir, loc, hic], axis=1).astype(_bf16)
    cd = cdr_ref[...]
    radial = cd[:, 4:5]
    ea_ext = jnp.concatenate(
        [ea_ref[...], jnp.broadcast_to(radial, (radial.shape[0], 8))],
        axis=1).astype(_bf16)
    a = jnp.dot(hcat, w1h_ref[...], preferred_element_type=_f32)
    a += jnp.dot(ea_ext, w1er_ref[...], preferred_element_type=_f32)
    a += bias_ref[0:1, :]
    m = _silu_bf(a)
    ef = jnp.dot(m, w2_ref[...], preferred_element_type=_f32)
    efb = _silu_bf(ef + bias_ref[1:2, :])
    av = lax.bitcast_convert_type(efb[:, :256], jnp.uint16).astype(jnp.uint32)
    bv = lax.bitcast_convert_type(efb[:, 256:], jnp.uint16).astype(jnp.uint32)
    ef_ref[...] = lax.bitcast_convert_type(av | (bv << 16), jnp.int32)
    cm = _silu_bf(jnp.dot(efb, wc1_ref[...],
                          preferred_element_type=_f32) + bias_ref[2:3, :])
    scale = jnp.dot(cm, wc2_ref[...],
                    preferred_element_type=_f32)[:, 0:1]
    t8 = cd * scale
    t16_ref[...] = (jnp.concatenate(
        [t8, jnp.zeros((t8.shape[0], 120), _f32)], axis=1)
        + bias_ref[3:4, 0:128])


def _tc_edge(hr, hc, cdr, ea, w1h, w1er, w2, wc1, wc2p, bias):
    B = 1000
    grid = (E // B,)
    return pl.pallas_call(
        _edge_kernel,
        grid=grid,
        in_specs=[
            pl.BlockSpec((B, D // 2), lambda i: (i, 0)),
            pl.BlockSpec((B, D // 2), lambda i: (i, 0)),
            pl.BlockSpec((B, 8), lambda i: (i, 0)),
            pl.BlockSpec((B, DE), lambda i: (i, 0)),
            pl.BlockSpec((H, H), lambda i: (0, 0)),
            pl.BlockSpec((24, H), lambda i: (0, 0)),
            pl.BlockSpec((H, H), lambda i: (0, 0)),
            pl.BlockSpec((H, H), lambda i: (0, 0)),
            pl.BlockSpec((H, 128), lambda i: (0, 0)),
            pl.BlockSpec((8, H), lambda i: (0, 0)),
        ],
        out_specs=[
            pl.BlockSpec((B, 256), lambda i: (i, 0)),
            pl.BlockSpec((B, 128), lambda i: (i, 0)),
        ],
        out_shape=[
            jax.ShapeDtypeStruct((E, 256), jnp.int32),
            jax.ShapeDtypeStruct((E, 128), _f32),
        ],
    )(hr, hc, cdr, ea, w1h, w1er, w2, wc1, wc2p, bias)


def _node_kernel(h_ref, nagg_ref, cagg_ref, cpad_ref,
                 w3a_ref, w3b_ref, w4_ref, bias_ref,
                 hout_ref, cout_ref):
    hb = h_ref[...].astype(_bf16)
    nb = nagg_ref[...].astype(_bf16)
    nhid = _silu_bf(jnp.dot(hb, w3a_ref[...], preferred_element_type=_f32)
                    + jnp.dot(nb, w3b_ref[...], preferred_element_type=_f32)
                    + bias_ref[0:1, :])
    nout = jnp.dot(nhid, w4_ref[...],
                   preferred_element_type=_f32) + bias_ref[1:2, 0:D]
    hout_ref[...] = h_ref[...] + nout
    cagg = cagg_ref[...][:, 0:16]
    cnt = jnp.clip(cagg[:, 3:4], 1.0, None)
    cout_ref[...] = cpad_ref[...] + cagg / cnt


def _tc_node(h, nagg, cagg, cpad, w3a, w3b, w4, bias2):
    B = 1000
    grid = (N // B,)
    return pl.pallas_call(
        _node_kernel,
        grid=grid,
        in_specs=[
            pl.BlockSpec((B, D), lambda i: (i, 0)),
            pl.BlockSpec((B, H), lambda i: (i, 0)),
            pl.BlockSpec((B, 128), lambda i: (i, 0)),
            pl.BlockSpec((B, 16), lambda i: (i, 0)),
            pl.BlockSpec((D, H), lambda i: (0, 0)),
            pl.BlockSpec((H, H), lambda i: (0, 0)),
            pl.BlockSpec((H, D), lambda i: (0, 0)),
            pl.BlockSpec((8, H), lambda i: (0, 0)),
        ],
        out_specs=[
            pl.BlockSpec((B, D), lambda i: (i, 0)),
            pl.BlockSpec((B, 16), lambda i: (i, 0)),
        ],
        out_shape=[
            jax.ShapeDtypeStruct((N, D), _f32),
            jax.ShapeDtypeStruct((N, 16), _f32),
        ],
    )(h, nagg, cagg, cpad, w3a, w3b, w4, bias2)


def kernel(h, edge_index, coord, edge_attr,
           W1, b1, W2, b2, W3, b3, W4, b4, Wc1, bc1, Wc2):
    row = edge_index[0]
    col = edge_index[1]
    cpad = jnp.pad(coord, ((0, 0), (0, 13)))
    cflat = coord.reshape(3 * N)
    hpk = lax.bitcast_convert_type(
        h.astype(_bf16).reshape(N, D // 2, 2), jnp.int32)

    wr, wc = W1[:D], W1[D:2 * D]
    w1h = jnp.concatenate([wr[0::2], wr[1::2], wc[0::2], wc[1::2]],
                          axis=0).astype(_bf16)
    w1er = jnp.zeros((24, H), _f32).at[:DE].set(W1[2 * D + 1:]) \
        .at[DE].set(W1[2 * D]).astype(_bf16)
    bias = jnp.zeros((8, H), _f32).at[0].set(b1).at[1].set(b2) \
        .at[2].set(bc1).at[3, 3].set(1.0)
    wc2p = jnp.zeros((H, 128), _f32).at[:, 0].set(Wc2[:, 0]).astype(_bf16)
    w3a = W3[:D].astype(_bf16)
    w3b = W3[D:].astype(_bf16)
    w4 = W4.astype(_bf16)
    bias2 = jnp.zeros((8, H), _f32).at[0].set(b3).at[1, :D].set(b4)

    hr, hc, cdr = _sc_gather(hpk, cflat, row, col)
    ef, t16 = _tc_edge(hr, hc, cdr, edge_attr,
                       w1h, w1er, W2.astype(_bf16), Wc1.astype(_bf16),
                       wc2p, bias)
    gef, gt, lid = _sc_regroup(row, ef, t16)
    lid3 = lid.reshape(NW * AGG_K, 1, AGG_B)
    nagg_pad, cagg_pad = _tc_agg(lid3, gef, gt)
    nagg = nagg_pad.reshape(NW, AGG_R, H)[:, :NRANGE].reshape(NW * NRANGE, H)
    cagg = cagg_pad.reshape(NW, AGG_R, 128)[:, :NRANGE]
    cagg = cagg.reshape(NW * NRANGE, 128)
    h_out, cout16 = _tc_node(h, nagg[:N], cagg[:N], cpad,
                             w3a, w3b, w4, bias2)
    return h_out, cout16[:, :3], edge_attr
